# baseline (speedup 1.0000x reference)
import functools

import jax
import jax.numpy as jnp
from jax import lax
from jax.experimental import pallas as pl
from jax.experimental.pallas import tpu as pltpu
from jax.experimental.pallas import tpu_sc as plsc

_EPS = 1e-5


def _round_up(v, m):
    return (v + m - 1) // m * m


@functools.partial(jax.jit, static_argnames=("N", "E", "D", "R"))
def _sc_aggregate(x, src, dst, et, N, E, D, R):
    H4 = _round_up(_round_up(N, 4) // 4, 8)
    S4 = R * H4
    ROWS = _round_up(S4 + 1, 128)
    PT = ROWS // 16
    _zchunks = [(k * 128, 128) for k in range(PT // 128)]
    if PT % 128:
        _zchunks.append((PT - PT % 128, PT % 128))
    EPT = E // 16
    MACRO = 4000
    NM = EPT // MACRO
    SUB = 128
    NB = 2
    IBR = (MACRO + SUB + 127) // 128 + 1
    assert EPT * 16 == E and NM * MACRO == EPT and MACRO % 16 == 0
    assert PT % 8 == 0 and all(z % 8 == 0 for _, z in _zchunks)

    mesh = plsc.VectorSubcoreMesh(core_axis_name="c", subcore_axis_name="s")

    @functools.partial(
        pl.kernel,
        mesh=mesh,
        compiler_params=pltpu.CompilerParams(needs_layout_passes=False),
        out_type=[
            jax.ShapeDtypeStruct((4, ROWS, D), jnp.float32),
            jax.ShapeDtypeStruct((4 * 16 * ROWS,), jnp.float32),
        ],
        scratch_types=[
            pltpu.VMEM_SHARED((ROWS, D), jnp.float32),
            pltpu.VMEM((MACRO,), jnp.int32),
            pltpu.VMEM((MACRO,), jnp.int32),
            pltpu.VMEM((MACRO,), jnp.int32),
            pltpu.VMEM((IBR * SUB,), jnp.int32),
            pltpu.VMEM((IBR, SUB), jnp.int32),
            pltpu.VMEM((NB, SUB, D), jnp.float32),
            pltpu.VMEM((ROWS,), jnp.float32),
            [pltpu.SemaphoreType.DMA] * NB,
            [pltpu.SemaphoreType.DMA] * NB,
            pltpu.SemaphoreType.DMA,
        ],
    )
    def agg(x_h, src_h, dst_h, et_h, acc_h, cnt_h,
            acc_sh, e_src, e_dst, e_et, idxg, idxa, rowb, cntloc,
            sems, ssems, sem_st):
        c = lax.axis_index("c")
        s = lax.axis_index("s")

        zero16 = jnp.zeros((16,), jnp.float32)
        one16 = jnp.full((16,), 1.0, jnp.float32)
        dum16 = jnp.full((16,), S4, jnp.int32)
        zeroi16 = jnp.zeros((16,), jnp.int32)
        iota16 = lax.iota(jnp.int32, 16)

        def fire(nsc):
            def step(u, c2):
                for b in range(NB):
                    @pl.when((u + NB - 1) % NB == b)
                    def _(b=b):
                        @pl.when((u >= 1) & (u - 1 < nsc))
                        def _():
                            pltpu.make_async_copy(
                                x_h.at[pl.ds(0, SUB)], rowb.at[b],
                                sems[b]).wait()
                            pltpu.async_copy(rowb.at[b],
                                             acc_sh.at[idxa.at[u - 1]],
                                             ssems[b], add=True)

                    @pl.when(u % NB == b)
                    def _(b=b):
                        @pl.when(u < nsc)
                        def _():
                            @pl.when(u >= NB)
                            def _():
                                pltpu.make_async_copy(
                                    rowb.at[b],
                                    acc_sh.at[pl.ds(0, SUB)],
                                    ssems[b]).wait()
                            pltpu.async_copy(
                                x_h.at[idxg.at[pl.ds(u * SUB, SUB)]],
                                rowb.at[b], sems[b])
                return c2

            lax.fori_loop(0, nsc + 1, step, 0)
            for k in range(NB):
                @pl.when(k < nsc)
                def _(k=k):
                    for b in range(NB):
                        @pl.when((nsc - 1 - k) % NB == b)
                        def _(b=b):
                            pltpu.make_async_copy(
                                rowb.at[b], acc_sh.at[pl.ds(0, SUB)],
                                ssems[b]).wait()

        for q in range(2):
            Q = 2 * q + c
            lo = Q * H4

            def fill_cnt(i, carry):
                cntloc[pl.ds(i * 16, 16)] = zero16
                return carry
            lax.fori_loop(0, ROWS // 16, fill_cnt, 0)

            def fill_rowb(i, carry):
                for j in range(D // 16):
                    rowb[0, i, pl.ds(j * 16, 16)] = zero16
                return carry
            lax.fori_loop(0, SUB, fill_rowb, 0)

            for zoff, zlen in _zchunks:
                pltpu.sync_copy(rowb.at[0, pl.ds(0, zlen)],
                                acc_sh.at[pl.ds(s * PT + zoff, zlen)])
            plsc.subcore_barrier()

            def stage(m):
                base = s * EPT + m * MACRO
                pltpu.async_copy(src_h.at[pl.ds(base, MACRO)], e_src,
                                 sem_st)
                pltpu.async_copy(dst_h.at[pl.ds(base, MACRO)], e_dst,
                                 sem_st)
                pltpu.async_copy(et_h.at[pl.ds(base, MACRO)], e_et,
                                 sem_st)

            stage(0)

            def macro(m, rem):
                for buf in (e_src, e_dst, e_et):
                    pltpu.make_async_copy(
                        src_h.at[pl.ds(0, MACRO)], buf, sem_st).wait()

                def compact(g, off2):
                    half = []
                    for h in range(2):
                        sl = pl.ds(g * 32 + h * 16, 16)
                        s16 = e_src[sl]
                        d16 = e_dst[sl]
                        t16 = e_et[sl]
                        inr = (d16 >= lo) & (d16 < lo + H4)
                        ia = t16 * H4 + d16 - lo
                        plsc.addupdate_scatter(cntloc, [ia], one16,
                                               mask=inr)
                        cs = plsc.cumsum(inr.astype(jnp.int32))
                        half.append((s16, ia, inr, cs))
                    for s16, ia, inr, cs in half:
                        pos = off2 + cs - 1
                        plsc.store_scatter(idxg, [pos], s16, mask=inr)
                        plsc.store_scatter(
                            idxa, [pos >> 7, pos & 127], ia, mask=inr)
                        off2 = off2 + jnp.max(cs)
                    return off2

                off = lax.fori_loop(0, MACRO // 32, compact, rem)

                @pl.when(m + 1 < NM)
                def _():
                    stage(m + 1)

                nfull = off >> 7
                fire(nfull)
                for k in range(SUB // 16):
                    vg = idxg[pl.ds(nfull * SUB + k * 16, 16)]
                    va = idxa[nfull, pl.ds(k * 16, 16)]
                    idxg[pl.ds(k * 16, 16)] = vg
                    idxa[0, pl.ds(k * 16, 16)] = va
                return off & 127

            rem = lax.fori_loop(0, NM, macro, jnp.int32(0))

            for k in range(SUB // 16):
                pv = rem + k * 16 + iota16
                idxg[pl.ds(rem + k * 16, 16)] = zeroi16
                plsc.store_scatter(idxa, [pv >> 7, pv & 127], dum16)

            @pl.when(rem > 0)
            def _():
                fire(jnp.int32(1))
            plsc.subcore_barrier()

            for zoff, zlen in _zchunks:
                pltpu.sync_copy(acc_sh.at[pl.ds(s * PT + zoff, zlen)],
                                acc_h.at[Q, pl.ds(s * PT + zoff, zlen)])
            pltpu.sync_copy(cntloc,
                            cnt_h.at[pl.ds((Q * 16 + s) * ROWS, ROWS)])
            plsc.subcore_barrier()

    return agg(x, src, dst, et)


def _post_body(R, H4, acc_r, cnt_r, x_r, wrel_r, wroot_r, wres_r,
               bias_r, bres_r, gamma_r, beta_r, out_r):
    a = acc_r[0]
    cn = jnp.sum(cnt_r[0], axis=0)
    xb = x_r[...]
    w = wroot_r[...] + wres_r[...]
    o = jnp.dot(xb, w, preferred_element_type=jnp.float32)
    for r in range(R):
        ar = a[r * H4:(r + 1) * H4, :]
        rr = 1.0 / jnp.maximum(cn[r * H4:(r + 1) * H4], 1.0)
        o = o + jnp.dot(ar * rr[:, None], wrel_r[r],
                        preferred_element_type=jnp.float32)
    o = o + bias_r[...] + bres_r[...]
    scale = gamma_r[...] * jax.lax.rsqrt(jnp.float32(1.0) + jnp.float32(_EPS))
    o = o * scale + beta_r[...]
    out_r[...] = jnp.maximum(o, 0.0)


def kernel(x, edge_index, edge_type, W_rel, W_root, bias, W_res, b_res,
           gamma, beta):
    N, D = x.shape
    E = edge_type.shape[0]
    R = W_rel.shape[0]
    H4 = _round_up(_round_up(N, 4) // 4, 8)
    ROWS = _round_up(R * H4 + 1, 128)

    src = edge_index[0]
    dst = edge_index[1]
    acc, cnt = _sc_aggregate(x, src, dst, edge_type, N=N, E=E, D=D, R=R)
    cnt3 = cnt.reshape(4, 16, ROWS)

    post = pl.pallas_call(
        functools.partial(_post_body, R, H4),
        grid=(4,),
        in_specs=[
            pl.BlockSpec((1, ROWS, D), lambda h: (h, 0, 0)),
            pl.BlockSpec((1, 16, ROWS), lambda h: (h, 0, 0)),
            pl.BlockSpec((H4, D), lambda h: (h, 0)),
            pl.BlockSpec((R, D, D), lambda h: (0, 0, 0)),
            pl.BlockSpec((D, D), lambda h: (0, 0)),
            pl.BlockSpec((D, D), lambda h: (0, 0)),
            pl.BlockSpec((1, D), lambda h: (0, 0)),
            pl.BlockSpec((1, D), lambda h: (0, 0)),
            pl.BlockSpec((1, D), lambda h: (0, 0)),
            pl.BlockSpec((1, D), lambda h: (0, 0)),
        ],
        out_specs=pl.BlockSpec((H4, D), lambda h: (h, 0)),
        out_shape=jax.ShapeDtypeStruct((N, D), jnp.float32),
    )
    return post(acc, cnt3, x, W_rel, W_root, W_res,
                bias.reshape(1, D), b_res.reshape(1, D),
                gamma.reshape(1, D), beta.reshape(1, D))


# speedup vs baseline: 1.1345x; 1.1345x over previous
import functools

import jax
import jax.numpy as jnp
from jax import lax
from jax.experimental import pallas as pl
from jax.experimental.pallas import tpu as pltpu
from jax.experimental.pallas import tpu_sc as plsc

_EPS = 1e-5


def _round_up(v, m):
    return (v + m - 1) // m * m


@functools.partial(jax.jit, static_argnames=("N", "E", "D", "R"))
def _sc_aggregate(x, src, dst, et, N, E, D, R):
    H4 = _round_up(_round_up(N, 4) // 4, 8)
    S4 = R * H4
    ROWS = _round_up(S4 + 1, 128)
    PT = ROWS // 16
    EPT = E // 16
    MACRO = 4000
    NM = EPT // MACRO
    SUB = 64
    NB = 4
    SH = SUB.bit_length() - 1
    IBR = (MACRO + 2 * SUB - 1) // SUB + 1
    _zchunks = [(k * SUB, SUB) for k in range(PT // SUB)]
    if PT % SUB:
        _zchunks.append((PT - PT % SUB, PT % SUB))
    assert EPT * 16 == E and NM * MACRO == EPT and MACRO % 16 == 0
    assert PT % 8 == 0 and all(z % 8 == 0 for _, z in _zchunks)

    mesh = plsc.VectorSubcoreMesh(core_axis_name="c", subcore_axis_name="s")

    @functools.partial(
        pl.kernel,
        mesh=mesh,
        compiler_params=pltpu.CompilerParams(needs_layout_passes=False),
        out_type=[
            jax.ShapeDtypeStruct((4, ROWS, D), jnp.float32),
            jax.ShapeDtypeStruct((4 * 16 * ROWS,), jnp.float32),
        ],
        scratch_types=[
            pltpu.VMEM_SHARED((ROWS, D), jnp.float32),
            pltpu.VMEM((MACRO,), jnp.int32),
            pltpu.VMEM((MACRO,), jnp.int32),
            pltpu.VMEM((MACRO,), jnp.int32),
            pltpu.VMEM((IBR * SUB,), jnp.int32),
            pltpu.VMEM((IBR, SUB), jnp.int32),
            pltpu.VMEM((NB, SUB, D), jnp.float32),
            pltpu.VMEM((ROWS,), jnp.float32),
            [pltpu.SemaphoreType.DMA] * NB,
            [pltpu.SemaphoreType.DMA] * NB,
            pltpu.SemaphoreType.DMA,
        ],
    )
    def agg(x_h, src_h, dst_h, et_h, acc_h, cnt_h,
            acc_sh, e_src, e_dst, e_et, idxg, idxa, rowb, cntloc,
            sems, ssems, sem_st):
        c = lax.axis_index("c")
        s = lax.axis_index("s")

        zero16 = jnp.zeros((16,), jnp.float32)
        one16 = jnp.full((16,), 1.0, jnp.float32)
        dum16 = jnp.full((16,), S4, jnp.int32)
        zeroi16 = jnp.zeros((16,), jnp.int32)
        iota16 = lax.iota(jnp.int32, 16)

        def fire(nsc):
            def step(u, c2):
                for b in range(NB):
                    @pl.when((u + NB - 1) % NB == b)
                    def _(b=b):
                        @pl.when((u >= 1) & (u - 1 < nsc))
                        def _():
                            pltpu.make_async_copy(
                                x_h.at[pl.ds(0, SUB)], rowb.at[b],
                                sems[b]).wait()
                            pltpu.async_copy(rowb.at[b],
                                             acc_sh.at[idxa.at[u - 1]],
                                             ssems[b], add=True)

                    @pl.when(u % NB == b)
                    def _(b=b):
                        @pl.when(u < nsc)
                        def _():
                            @pl.when(u >= NB)
                            def _():
                                pltpu.make_async_copy(
                                    rowb.at[b],
                                    acc_sh.at[pl.ds(0, SUB)],
                                    ssems[b]).wait()
                            pltpu.async_copy(
                                x_h.at[idxg.at[pl.ds(u * SUB, SUB)]],
                                rowb.at[b], sems[b])
                return c2

            lax.fori_loop(0, nsc + 1, step, 0)
            for k in range(NB):
                @pl.when(k < nsc)
                def _(k=k):
                    for b in range(NB):
                        @pl.when((nsc - 1 - k) % NB == b)
                        def _(b=b):
                            pltpu.make_async_copy(
                                rowb.at[b], acc_sh.at[pl.ds(0, SUB)],
                                ssems[b]).wait()

        for q in range(2):
            Q = 2 * q + c
            lo = Q * H4

            def fill_cnt(i, carry):
                cntloc[pl.ds(i * 16, 16)] = zero16
                return carry
            lax.fori_loop(0, ROWS // 16, fill_cnt, 0)

            def fill_rowb(i, carry):
                for j in range(D // 16):
                    rowb[0, i, pl.ds(j * 16, 16)] = zero16
                return carry
            lax.fori_loop(0, SUB, fill_rowb, 0)

            for zoff, zlen in _zchunks:
                pltpu.sync_copy(rowb.at[0, pl.ds(0, zlen)],
                                acc_sh.at[pl.ds(s * PT + zoff, zlen)])
            plsc.subcore_barrier()

            def stage(m):
                base = s * EPT + m * MACRO
                pltpu.async_copy(src_h.at[pl.ds(base, MACRO)], e_src,
                                 sem_st)
                pltpu.async_copy(dst_h.at[pl.ds(base, MACRO)], e_dst,
                                 sem_st)
                pltpu.async_copy(et_h.at[pl.ds(base, MACRO)], e_et,
                                 sem_st)

            stage(0)

            def macro(m, rem):
                for buf in (e_src, e_dst, e_et):
                    pltpu.make_async_copy(
                        src_h.at[pl.ds(0, MACRO)], buf, sem_st).wait()

                def compact(g, off2):
                    half = []
                    for h in range(2):
                        sl = pl.ds(g * 32 + h * 16, 16)
                        s16 = e_src[sl]
                        d16 = e_dst[sl]
                        t16 = e_et[sl]
                        inr = (d16 >= lo) & (d16 < lo + H4)
                        ia = t16 * H4 + d16 - lo
                        plsc.addupdate_scatter(cntloc, [ia], one16,
                                               mask=inr)
                        cs = plsc.cumsum(inr.astype(jnp.int32))
                        half.append((s16, ia, inr, cs))
                    for s16, ia, inr, cs in half:
                        pos = off2 + cs - 1
                        plsc.store_scatter(idxg, [pos], s16, mask=inr)
                        plsc.store_scatter(
                            idxa, [pos >> SH, pos & (SUB - 1)], ia,
                            mask=inr)
                        off2 = off2 + jnp.max(cs)
                    return off2

                off = lax.fori_loop(0, MACRO // 32, compact, rem)

                @pl.when(m + 1 < NM)
                def _():
                    stage(m + 1)

                nfull = off >> SH
                fire(nfull)
                for k in range(SUB // 16):
                    vg = idxg[pl.ds(nfull * SUB + k * 16, 16)]
                    va = idxa[nfull, pl.ds(k * 16, 16)]
                    idxg[pl.ds(k * 16, 16)] = vg
                    idxa[0, pl.ds(k * 16, 16)] = va
                return off & (SUB - 1)

            rem = lax.fori_loop(0, NM, macro, jnp.int32(0))

            for k in range(SUB // 16):
                pv = rem + k * 16 + iota16
                idxg[pl.ds(rem + k * 16, 16)] = zeroi16
                plsc.store_scatter(idxa, [pv >> SH, pv & (SUB - 1)], dum16)

            @pl.when(rem > 0)
            def _():
                fire(jnp.int32(1))
            plsc.subcore_barrier()

            for zoff, zlen in _zchunks:
                pltpu.sync_copy(acc_sh.at[pl.ds(s * PT + zoff, zlen)],
                                acc_h.at[Q, pl.ds(s * PT + zoff, zlen)])
            pltpu.sync_copy(cntloc,
                            cnt_h.at[pl.ds((Q * 16 + s) * ROWS, ROWS)])
            plsc.subcore_barrier()

    return agg(x, src, dst, et)


def _post_body(R, H4, acc_r, cnt_r, x_r, wrel_r, wroot_r, wres_r,
               bias_r, bres_r, gamma_r, beta_r, out_r):
    a = acc_r[0]
    cn = jnp.sum(cnt_r[0], axis=0)
    xb = x_r[...]
    w = wroot_r[...] + wres_r[...]
    o = jnp.dot(xb, w, preferred_element_type=jnp.float32)
    for r in range(R):
        ar = a[r * H4:(r + 1) * H4, :]
        rr = 1.0 / jnp.maximum(cn[r * H4:(r + 1) * H4], 1.0)
        o = o + jnp.dot(ar * rr[:, None], wrel_r[r],
                        preferred_element_type=jnp.float32)
    o = o + bias_r[...] + bres_r[...]
    scale = gamma_r[...] * jax.lax.rsqrt(jnp.float32(1.0) + jnp.float32(_EPS))
    o = o * scale + beta_r[...]
    out_r[...] = jnp.maximum(o, 0.0)


def kernel(x, edge_index, edge_type, W_rel, W_root, bias, W_res, b_res,
           gamma, beta):
    N, D = x.shape
    E = edge_type.shape[0]
    R = W_rel.shape[0]
    H4 = _round_up(_round_up(N, 4) // 4, 8)
    ROWS = _round_up(R * H4 + 1, 128)

    src = edge_index[0]
    dst = edge_index[1]
    acc, cnt = _sc_aggregate(x, src, dst, edge_type, N=N, E=E, D=D, R=R)
    cnt3 = cnt.reshape(4, 16, ROWS)

    post = pl.pallas_call(
        functools.partial(_post_body, R, H4),
        grid=(4,),
        in_specs=[
            pl.BlockSpec((1, ROWS, D), lambda h: (h, 0, 0)),
            pl.BlockSpec((1, 16, ROWS), lambda h: (h, 0, 0)),
            pl.BlockSpec((H4, D), lambda h: (h, 0)),
            pl.BlockSpec((R, D, D), lambda h: (0, 0, 0)),
            pl.BlockSpec((D, D), lambda h: (0, 0)),
            pl.BlockSpec((D, D), lambda h: (0, 0)),
            pl.BlockSpec((1, D), lambda h: (0, 0)),
            pl.BlockSpec((1, D), lambda h: (0, 0)),
            pl.BlockSpec((1, D), lambda h: (0, 0)),
            pl.BlockSpec((1, D), lambda h: (0, 0)),
        ],
        out_specs=pl.BlockSpec((H4, D), lambda h: (h, 0)),
        out_shape=jax.ShapeDtypeStruct((N, D), jnp.float32),
    )
    return post(acc, cnt3, x, W_rel, W_root, W_res,
                bias.reshape(1, D), b_res.reshape(1, D),
                gamma.reshape(1, D), beta.reshape(1, D))
